# Initial kernel scaffold; baseline (speedup 1.0000x reference)
import jax
import jax.numpy as jnp
from jax.experimental import pallas as pl


def kernel(problems):
    raise NotImplementedError("write your pallas kernel here")


# speedup vs baseline: 1.4608x; 1.4608x over previous
import jax
import jax.numpy as jnp
from jax import lax
from jax.experimental import pallas as pl
from jax.experimental.pallas import tpu as pltpu
from jax.experimental.pallas import tpu_sc as plsc

_SEED_CNT = 128
_EMBED_DIM = 256
_MACHINE_CNT = 16
_LANES = 16
_NUM_WORKERS = 32
_BLOCK = _MACHINE_CNT * _EMBED_DIM


def _col_body(rand_hbm, col_hbm, rand_v, block_v):
    rows_per_w = rand_hbm.shape[0] // _SEED_CNT // _NUM_WORKERS
    wid = lax.axis_index("s") * 2 + lax.axis_index("c")
    base = wid * rows_per_w
    pltpu.sync_copy(rand_hbm.at[pl.ds(base * _SEED_CNT, rows_per_w * _SEED_CNT)],
                    rand_v)

    iota = lax.iota(jnp.int32, _LANES)
    ones = jnp.ones((_LANES,), jnp.float32)
    zeros = jnp.zeros((_LANES,), jnp.float32)
    machine_off = iota * _EMBED_DIM

    def zero_init(j, carry):
        block_v[pl.ds(j * _LANES, _LANES)] = zeros
        return carry

    lax.fori_loop(0, _BLOCK // _LANES, zero_init, 0)

    def per_batch(i, carry):
        cur = None
        for j in range(_SEED_CNT // _LANES):
            v = rand_v[pl.ds(i * _SEED_CNT + j * _LANES, _LANES)]
            k = (v * 8388608.0).astype(jnp.int32) * _SEED_CNT + (iota + j * _LANES)
            s, _ = plsc.sort_key_val(k, k)
            if cur is None:
                cur = s
            else:
                m = jnp.minimum(cur, lax.rev(s, (0,)))
                cur, _ = plsc.sort_key_val(m, m)
        idx = lax.bitwise_and(cur, _SEED_CNT - 1)
        offs = machine_off + idx
        plsc.store_scatter(block_v, [offs], ones)
        pltpu.sync_copy(block_v, col_hbm.at[pl.ds((base + i) * _BLOCK, _BLOCK)])
        plsc.store_scatter(block_v, [offs], zeros)
        return carry

    lax.fori_loop(0, rows_per_w, per_batch, 0)


def _make_col_kernel(batch_size):
    rows_per_w = batch_size // _NUM_WORKERS
    mesh = plsc.VectorSubcoreMesh(core_axis_name="c", subcore_axis_name="s")
    return pl.kernel(
        _col_body,
        out_type=jax.ShapeDtypeStruct((batch_size * _BLOCK,), jnp.float32),
        mesh=mesh,
        compiler_params=pltpu.CompilerParams(needs_layout_passes=False),
        scratch_types=[
            pltpu.VMEM((rows_per_w * _SEED_CNT,), jnp.float32),
            pltpu.VMEM((_BLOCK,), jnp.float32),
        ],
    )


def kernel(problems):
    batch_size, job_cnt, machine_cnt = problems.shape
    assert machine_cnt == _MACHINE_CNT and batch_size % _NUM_WORKERS == 0
    rand = jax.random.uniform(jax.random.key(42), (batch_size, _SEED_CNT),
                              dtype=jnp.float32)
    col_flat = _make_col_kernel(batch_size)(rand.reshape(-1))
    col_emb = col_flat.reshape(batch_size, _MACHINE_CNT, _EMBED_DIM)
    row_emb = jnp.zeros((batch_size, job_cnt, _EMBED_DIM), dtype=jnp.float32)
    return (row_emb, col_emb)
